# baseline (speedup 1.0000x reference)
import functools

import jax
import jax.numpy as jnp
from jax import lax
from jax.experimental import pallas as pl
from jax.experimental.pallas import tpu as pltpu
from jax.experimental.pallas import tpu_sc as plsc

_NUM_CORES = 2
_NUM_SUBCORES = 16
_NUM_WORKERS = _NUM_CORES * _NUM_SUBCORES

_VOCAB = 1000000
_EMBED = 32
_VCOLS = _VOCAB // 128
_VMAIN = _VCOLS * 128
_T2_ROWS = _VOCAB * _EMBED // 128

_DCOLS = 4
_DUNITS = _VCOLS // _DCOLS
_DSLOTS = 62

_UNITS = 6400
_K = 4
_GSTEPS = _UNITS // (_NUM_WORKERS * _K)

_mesh = plsc.VectorSubcoreMesh(core_axis_name="c", subcore_axis_name="s")


def _wid():
  return lax.axis_index("s") * _NUM_CORES + lax.axis_index("c")


@functools.partial(
    pl.kernel,
    out_type=jax.ShapeDtypeStruct((_T2_ROWS, 128), jnp.float32),
    mesh=_mesh,
    scratch_types=[
        pltpu.VMEM((32, 128 * _DCOLS), jnp.float32),
        pltpu.VMEM((32, 128 * _DCOLS), jnp.float32),
        pltpu.VMEM((32, 128 * _DCOLS + 1), jnp.float32),
        pltpu.VMEM((32 * _DCOLS, 128), jnp.float32),
        pltpu.VMEM((32 * _DCOLS, 128), jnp.float32),
        pltpu.VMEM((16, 128), jnp.float32),
        pltpu.SemaphoreType.DMA,
        pltpu.SemaphoreType.DMA,
    ],
    compiler_params=pltpu.CompilerParams(
        use_tc_tiling_on_sc=True, needs_layout_passes=False
    ),
)
def _detile(tt_hbm, tail_hbm, t2_hbm, in0, in1, inp, out0, out1, tail_t,
            insem, outsem):
  w = _wid()
  lanes = lax.iota(jnp.int32, 16)
  ins = (in0, in1)
  outs = (out0, out1)
  width = 128 * _DCOLS

  def valid(t):
    return (w + _NUM_WORKERS * t) < _DUNITS

  def fire_in(t, buf):
    u = w + _NUM_WORKERS * t
    for dt in range(4):
      pltpu.async_copy(
          tt_hbm.at[pl.ds(dt * 8, 8), pl.ds(u * width, width)],
          buf.at[pl.ds(dt * 8, 8)],
          insem,
      )

  def transpose(buf_in, buf_out):
    def rbody(d, carry):
      for i in range(128 * _DCOLS // 16):
        inp[d, pl.ds(16 * i, 16)] = buf_in[d, pl.ds(16 * i, 16)]
      return carry

    lax.fori_loop(0, 32, rbody, 0, unroll=2)

    def tbody(k, carry):
      for j in range(8):
        d_vec = lanes + 16 * (j % 2)
        c_vec = jnp.full((16,), 4 * k + j // 2, jnp.int32)
        buf_out[k, pl.ds(16 * j, 16)] = plsc.load_gather(inp, [d_vec, c_vec])
      return carry

    lax.fori_loop(0, 32 * _DCOLS, tbody, 0, unroll=8)

  fire_in(0, ins[0])

  def pair_body(p, carry):
    for b in range(2):
      t = 2 * p + b
      bin_, bout = ins[b], outs[b]

      @pl.when(valid(t))
      def _():
        pltpu.make_async_copy(
            tt_hbm.at[pl.ds(0, 32), pl.ds(0, width)], bin_, insem
        ).wait()

      @pl.when(valid(t + 1))
      def _():
        fire_in(t + 1, ins[1 - b])

      @pl.when((t >= 2) & valid(t - 2))
      def _():
        pltpu.make_async_copy(
            bout, t2_hbm.at[pl.ds(0, 32 * _DCOLS)], outsem
        ).wait()

      transpose(bin_, bout)

      @pl.when(valid(t))
      def _():
        u = w + _NUM_WORKERS * t
        pltpu.async_copy(
            bout, t2_hbm.at[pl.ds(u * 32 * _DCOLS, 32 * _DCOLS)], outsem
        )
    return carry

  lax.fori_loop(0, _DSLOTS // 2, pair_body, 0, unroll=False)

  @pl.when(valid(_DSLOTS - 2))
  def _():
    pltpu.make_async_copy(
        outs[0], t2_hbm.at[pl.ds(0, 32 * _DCOLS)], outsem
    ).wait()

  @pl.when(valid(_DSLOTS - 1))
  def _():
    pltpu.make_async_copy(
        outs[1], t2_hbm.at[pl.ds(0, 32 * _DCOLS)], outsem
    ).wait()

  @pl.when(w == 0)
  def _():
    pltpu.sync_copy(tail_hbm, tail_t)
    pltpu.sync_copy(tail_t, t2_hbm.at[pl.ds(_VMAIN * _EMBED // 128, 16)])


@functools.partial(
    pl.kernel,
    out_type=jax.ShapeDtypeStruct((_UNITS * 32, 128), jnp.float32),
    mesh=_mesh,
    scratch_types=[
        pltpu.VMEM((_K * 128,), jnp.int32),
        pltpu.VMEM((_K * 128,), jnp.int32),
        pltpu.VMEM((_K * 128, _EMBED), jnp.float32),
        pltpu.VMEM((_K * 128, _EMBED), jnp.float32),
        pltpu.VMEM((_K * 128, _EMBED + 1), jnp.float32),
        pltpu.VMEM((_K * 32, 128), jnp.float32),
        pltpu.VMEM((_K * 32, 128), jnp.float32),
        pltpu.SemaphoreType.DMA,
        pltpu.SemaphoreType.DMA,
        pltpu.SemaphoreType.DMA,
    ],
    compiler_params=pltpu.CompilerParams(
        use_tc_tiling_on_sc=False, needs_layout_passes=False
    ),
)
def _gather(t2_hbm, xa_hbm, out_hbm, idx0, idx1, g0, g1, gp, tr0, tr1,
            isem, gsem, ssem):
  w = _wid()
  lanes = lax.iota(jnp.int32, 16)
  idxs = (idx0, idx1)
  gs = (g0, g1)
  trs = (tr0, tr1)
  mbase = w * (_K * _GSTEPS)

  def fire_idx(s, buf):
    pltpu.async_copy(
        xa_hbm.at[pl.ds((mbase + _K * s) * 128, _K * 128)], buf, isem
    )

  def fire_stores(s, tr):
    m0 = mbase + _K * s
    for r in range(_K):
      m = m0 + r
      ht = m // 256
      bt = (m % 256) // 8
      hr = m % 8
      base = (8 * ht + hr) * 1024 + bt * 8
      for dt in range(4):
        pltpu.async_copy(
            tr.at[pl.ds(r * 32 + 8 * dt, 8)],
            out_hbm.at[pl.ds(base + dt * 256, 8)],
            ssem,
        )

  fire_idx(0, idxs[0])
  pltpu.make_async_copy(
      xa_hbm.at[pl.ds(0, _K * 128)], idxs[0], isem
  ).wait()
  pltpu.async_copy(t2_hbm.at[idxs[0]], gs[0], gsem)
  fire_idx(1, idxs[1])

  def pair_body(p, carry):
    for b in range(2):
      s = 2 * p + b
      g_v, tr_v = gs[b], trs[b]
      pltpu.make_async_copy(t2_hbm.at[idxs[b]], g_v, gsem).wait()

      @pl.when(s + 1 < _GSTEPS)
      def _():
        pltpu.make_async_copy(
            xa_hbm.at[pl.ds(0, _K * 128)], idxs[1 - b], isem
        ).wait()
        pltpu.async_copy(t2_hbm.at[idxs[1 - b]], gs[1 - b], gsem)

      @pl.when(s + 2 < _GSTEPS)
      def _():
        fire_idx(s + 2, idxs[b])

      @pl.when(s >= 2)
      def _():
        pltpu.make_async_copy(
            tr_v, out_hbm.at[pl.ds(0, _K * 32)], ssem
        ).wait()

      def rbody(q, carry):
        gp[q, pl.ds(0, 16)] = g_v[q, pl.ds(0, 16)]
        gp[q, pl.ds(16, 16)] = g_v[q, pl.ds(16, 16)]
        return carry

      lax.fori_loop(0, 128 * _K, rbody, 0, unroll=8)

      def tbody(q, carry):
        d_vec = jnp.full((16,), q % 32, jnp.int32)
        cbase = 128 * (q // 32)
        for j in range(8):
          c_vec = lanes + (cbase + 16 * j)
          tr_v[q, pl.ds(16 * j, 16)] = plsc.load_gather(gp, [c_vec, d_vec])
        return carry

      lax.fori_loop(0, 32 * _K, tbody, 0, unroll=8)
      fire_stores(s, tr_v)
    return carry

  lax.fori_loop(0, _GSTEPS // 2, pair_body, 0, unroll=False)
  pltpu.make_async_copy(trs[0], out_hbm.at[pl.ds(0, _K * 32)], ssem).wait()
  pltpu.make_async_copy(trs[1], out_hbm.at[pl.ds(0, _K * 32)], ssem).wait()


def kernel(x, table):
  batch, hist = x.shape
  tt = jnp.transpose(table)
  xa = (
      jnp.transpose(x)
      .reshape(hist // 8, 8, batch // 128, 128)
      .transpose(0, 2, 1, 3)
      .reshape(_UNITS * 128)
      .astype(jnp.int32)
  )
  tail = lax.slice(table, (_VMAIN, 0), (_VOCAB, _EMBED)).reshape(16, 128)

  t2 = _detile(tt, tail)
  t2v = t2.reshape(_VOCAB, _EMBED)
  a = _gather(t2v, xa)
  out = (
      a.reshape(hist, 4, batch // 128, 8, 128)
      .transpose(2, 4, 0, 1, 3)
      .reshape(batch, hist, _EMBED)
  )
  return out


# speedup vs baseline: 1.0002x; 1.0002x over previous
import functools

import jax
import jax.numpy as jnp
from jax import lax
from jax.experimental import pallas as pl
from jax.experimental.pallas import tpu as pltpu
from jax.experimental.pallas import tpu_sc as plsc

_NUM_CORES = 2
_NUM_SUBCORES = 16
_NUM_WORKERS = _NUM_CORES * _NUM_SUBCORES

_VOCAB = 1000000
_EMBED = 32
_VCOLS = _VOCAB // 128
_VMAIN = _VCOLS * 128
_T2_ROWS = _VOCAB * _EMBED // 128

_DCOLS = 4
_DUNITS = _VCOLS // _DCOLS
_DSLOTS = 62

_UNITS = 6400
_K = 4
_GSTEPS = _UNITS // (_NUM_WORKERS * _K)

_mesh = plsc.VectorSubcoreMesh(core_axis_name="c", subcore_axis_name="s")


def _wid():
  return lax.axis_index("s") * _NUM_CORES + lax.axis_index("c")


@functools.partial(
    pl.kernel,
    out_type=jax.ShapeDtypeStruct((_T2_ROWS, 128), jnp.float32),
    mesh=_mesh,
    scratch_types=[
        pltpu.VMEM((32, 128 * _DCOLS), jnp.float32),
        pltpu.VMEM((32, 128 * _DCOLS), jnp.float32),
        pltpu.VMEM((32, 128 * _DCOLS + 1), jnp.float32),
        pltpu.VMEM((32 * _DCOLS, 128), jnp.float32),
        pltpu.VMEM((32 * _DCOLS, 128), jnp.float32),
        pltpu.VMEM((16, 128), jnp.float32),
        pltpu.SemaphoreType.DMA,
        pltpu.SemaphoreType.DMA,
    ],
    compiler_params=pltpu.CompilerParams(
        use_tc_tiling_on_sc=True, needs_layout_passes=False,
        disable_bounds_checks=True
    ),
)
def _detile(tt_hbm, tail_hbm, t2_hbm, in0, in1, inp, out0, out1, tail_t,
            insem, outsem):
  w = _wid()
  lanes = lax.iota(jnp.int32, 16)
  ins = (in0, in1)
  outs = (out0, out1)
  width = 128 * _DCOLS

  def valid(t):
    return (w + _NUM_WORKERS * t) < _DUNITS

  def fire_in(t, buf):
    u = w + _NUM_WORKERS * t
    for dt in range(4):
      pltpu.async_copy(
          tt_hbm.at[pl.ds(dt * 8, 8), pl.ds(u * width, width)],
          buf.at[pl.ds(dt * 8, 8)],
          insem,
      )

  def transpose(buf_in, buf_out):
    def rbody(d, carry):
      for i in range(128 * _DCOLS // 16):
        inp[d, pl.ds(16 * i, 16)] = buf_in[d, pl.ds(16 * i, 16)]
      return carry

    lax.fori_loop(0, 32, rbody, 0, unroll=2)

    def tbody(k, carry):
      for j in range(8):
        d_vec = lanes + 16 * (j % 2)
        c_vec = jnp.full((16,), 4 * k + j // 2, jnp.int32)
        buf_out[k, pl.ds(16 * j, 16)] = plsc.load_gather(inp, [d_vec, c_vec])
      return carry

    lax.fori_loop(0, 32 * _DCOLS, tbody, 0, unroll=8)

  fire_in(0, ins[0])

  def pair_body(p, carry):
    for b in range(2):
      t = 2 * p + b
      bin_, bout = ins[b], outs[b]

      @pl.when(valid(t))
      def _():
        pltpu.make_async_copy(
            tt_hbm.at[pl.ds(0, 32), pl.ds(0, width)], bin_, insem
        ).wait()

      @pl.when(valid(t + 1))
      def _():
        fire_in(t + 1, ins[1 - b])

      @pl.when((t >= 2) & valid(t - 2))
      def _():
        pltpu.make_async_copy(
            bout, t2_hbm.at[pl.ds(0, 32 * _DCOLS)], outsem
        ).wait()

      transpose(bin_, bout)

      @pl.when(valid(t))
      def _():
        u = w + _NUM_WORKERS * t
        pltpu.async_copy(
            bout, t2_hbm.at[pl.ds(u * 32 * _DCOLS, 32 * _DCOLS)], outsem
        )
    return carry

  lax.fori_loop(0, _DSLOTS // 2, pair_body, 0, unroll=False)

  @pl.when(valid(_DSLOTS - 2))
  def _():
    pltpu.make_async_copy(
        outs[0], t2_hbm.at[pl.ds(0, 32 * _DCOLS)], outsem
    ).wait()

  @pl.when(valid(_DSLOTS - 1))
  def _():
    pltpu.make_async_copy(
        outs[1], t2_hbm.at[pl.ds(0, 32 * _DCOLS)], outsem
    ).wait()

  @pl.when(w == 0)
  def _():
    pltpu.sync_copy(tail_hbm, tail_t)
    pltpu.sync_copy(tail_t, t2_hbm.at[pl.ds(_VMAIN * _EMBED // 128, 16)])


@functools.partial(
    pl.kernel,
    out_type=jax.ShapeDtypeStruct((_UNITS * 32, 128), jnp.float32),
    mesh=_mesh,
    scratch_types=[
        pltpu.VMEM((_K * 128,), jnp.int32),
        pltpu.VMEM((_K * 128,), jnp.int32),
        pltpu.VMEM((_K * 128, _EMBED), jnp.float32),
        pltpu.VMEM((_K * 128, _EMBED), jnp.float32),
        pltpu.VMEM((_K * 128, _EMBED + 1), jnp.float32),
        pltpu.VMEM((_K * 32, 128), jnp.float32),
        pltpu.VMEM((_K * 32, 128), jnp.float32),
        pltpu.SemaphoreType.DMA,
        pltpu.SemaphoreType.DMA,
        pltpu.SemaphoreType.DMA,
    ],
    compiler_params=pltpu.CompilerParams(
        use_tc_tiling_on_sc=False, needs_layout_passes=False,
        disable_bounds_checks=True
    ),
)
def _gather(t2_hbm, xa_hbm, out_hbm, idx0, idx1, g0, g1, gp, tr0, tr1,
            isem, gsem, ssem):
  w = _wid()
  lanes = lax.iota(jnp.int32, 16)
  idxs = (idx0, idx1)
  gs = (g0, g1)
  trs = (tr0, tr1)
  mbase = w * (_K * _GSTEPS)

  def fire_idx(s, buf):
    pltpu.async_copy(
        xa_hbm.at[pl.ds((mbase + _K * s) * 128, _K * 128)], buf, isem
    )

  def fire_stores(s, tr):
    m0 = mbase + _K * s
    for r in range(_K):
      m = m0 + r
      ht = m // 256
      bt = (m % 256) // 8
      hr = m % 8
      base = (8 * ht + hr) * 1024 + bt * 8
      for dt in range(4):
        pltpu.async_copy(
            tr.at[pl.ds(r * 32 + 8 * dt, 8)],
            out_hbm.at[pl.ds(base + dt * 256, 8)],
            ssem,
        )

  fire_idx(0, idxs[0])
  pltpu.make_async_copy(
      xa_hbm.at[pl.ds(0, _K * 128)], idxs[0], isem
  ).wait()
  pltpu.async_copy(t2_hbm.at[idxs[0]], gs[0], gsem)
  fire_idx(1, idxs[1])

  def pair_body(p, carry):
    for b in range(2):
      s = 2 * p + b
      g_v, tr_v = gs[b], trs[b]
      pltpu.make_async_copy(t2_hbm.at[idxs[b]], g_v, gsem).wait()

      @pl.when(s + 1 < _GSTEPS)
      def _():
        pltpu.make_async_copy(
            xa_hbm.at[pl.ds(0, _K * 128)], idxs[1 - b], isem
        ).wait()
        pltpu.async_copy(t2_hbm.at[idxs[1 - b]], gs[1 - b], gsem)

      @pl.when(s + 2 < _GSTEPS)
      def _():
        fire_idx(s + 2, idxs[b])

      @pl.when(s >= 2)
      def _():
        pltpu.make_async_copy(
            tr_v, out_hbm.at[pl.ds(0, _K * 32)], ssem
        ).wait()

      def rbody(q, carry):
        gp[q, pl.ds(0, 16)] = g_v[q, pl.ds(0, 16)]
        gp[q, pl.ds(16, 16)] = g_v[q, pl.ds(16, 16)]
        return carry

      lax.fori_loop(0, 128 * _K, rbody, 0, unroll=8)

      def tbody(q, carry):
        d_vec = jnp.full((16,), q % 32, jnp.int32)
        cbase = 128 * (q // 32)
        for j in range(8):
          c_vec = lanes + (cbase + 16 * j)
          tr_v[q, pl.ds(16 * j, 16)] = plsc.load_gather(gp, [c_vec, d_vec])
        return carry

      lax.fori_loop(0, 32 * _K, tbody, 0, unroll=8)
      fire_stores(s, tr_v)
    return carry

  lax.fori_loop(0, _GSTEPS // 2, pair_body, 0, unroll=False)
  pltpu.make_async_copy(trs[0], out_hbm.at[pl.ds(0, _K * 32)], ssem).wait()
  pltpu.make_async_copy(trs[1], out_hbm.at[pl.ds(0, _K * 32)], ssem).wait()


def kernel(x, table):
  batch, hist = x.shape
  tt = jnp.transpose(table)
  xa = (
      jnp.transpose(x)
      .reshape(hist // 8, 8, batch // 128, 128)
      .transpose(0, 2, 1, 3)
      .reshape(_UNITS * 128)
      .astype(jnp.int32)
  )
  tail = lax.slice(table, (_VMAIN, 0), (_VOCAB, _EMBED)).reshape(16, 128)

  t2 = _detile(tt, tail)
  t2v = t2.reshape(_VOCAB, _EMBED)
  a = _gather(t2v, xa)
  out = (
      a.reshape(hist, 4, batch // 128, 8, 128)
      .transpose(2, 4, 0, 1, 3)
      .reshape(batch, hist, _EMBED)
  )
  return out


# speedup vs baseline: 1.6496x; 1.6493x over previous
import functools

import jax
import jax.numpy as jnp
from jax import lax
from jax.experimental import pallas as pl
from jax.experimental.pallas import tpu as pltpu
from jax.experimental.pallas import tpu_sc as plsc

_NUM_CORES = 2
_NUM_SUBCORES = 16
_NUM_WORKERS = _NUM_CORES * _NUM_SUBCORES

_VOCAB = 1000000
_EMBED = 32
_VCOLS = _VOCAB // 128
_VMAIN = _VCOLS * 128
_T2_ROWS = _VOCAB * _EMBED // 128

_DCOLS = 4
_DUNITS = _VCOLS // _DCOLS
_DSLOTS = 62

_UNITS = 6400
_K = 4
_GSTEPS = _UNITS // (_NUM_WORKERS * _K)

_mesh = plsc.VectorSubcoreMesh(core_axis_name="c", subcore_axis_name="s")


def _wid():
  return lax.axis_index("s") * _NUM_CORES + lax.axis_index("c")


@functools.partial(
    pl.kernel,
    out_type=jax.ShapeDtypeStruct((_T2_ROWS, 128), jnp.float32),
    mesh=_mesh,
    scratch_types=[
        pltpu.VMEM((32, 128 * _DCOLS), jnp.float32),
        pltpu.VMEM((32, 128 * _DCOLS), jnp.float32),
        pltpu.VMEM((32 * (128 * _DCOLS + 1),), jnp.float32),
        pltpu.VMEM((32 * _DCOLS, 128), jnp.float32),
        pltpu.VMEM((32 * _DCOLS, 128), jnp.float32),
        pltpu.VMEM((16, 128), jnp.float32),
        pltpu.SemaphoreType.DMA,
        pltpu.SemaphoreType.DMA,
    ],
    compiler_params=pltpu.CompilerParams(
        use_tc_tiling_on_sc=True, needs_layout_passes=False,
        disable_bounds_checks=True
    ),
)
def _detile(tt_hbm, tail_hbm, t2_hbm, in0, in1, inp, out0, out1, tail_t,
            insem, outsem):
  w = _wid()
  lanes = lax.iota(jnp.int32, 16)
  ins = (in0, in1)
  outs = (out0, out1)
  width = 128 * _DCOLS

  def valid(t):
    return (w + _NUM_WORKERS * t) < _DUNITS

  def fire_in(t, buf):
    u = w + _NUM_WORKERS * t
    for dt in range(4):
      pltpu.async_copy(
          tt_hbm.at[pl.ds(dt * 8, 8), pl.ds(u * width, width)],
          buf.at[pl.ds(dt * 8, 8)],
          insem,
      )

  stride = 128 * _DCOLS + 1
  lanes_s = lanes * stride

  def transpose(buf_in, buf_out):
    def rbody(d, carry):
      for i in range(128 * _DCOLS // 16):
        inp[pl.ds(d * stride + 16 * i, 16)] = buf_in[d, pl.ds(16 * i, 16)]
      return carry

    lax.fori_loop(0, 32, rbody, 0, unroll=2)

    def tbody(k, carry):
      av = lanes_s + 4 * k
      for j in range(8):
        buf_out[k, pl.ds(16 * j, 16)] = plsc.load_gather(inp, [av])
        av = av + (16 * stride if j % 2 == 0 else 1 - 16 * stride)
      return carry

    lax.fori_loop(0, 32 * _DCOLS, tbody, 0, unroll=8)

  fire_in(0, ins[0])

  def pair_body(p, carry):
    for b in range(2):
      t = 2 * p + b
      bin_, bout = ins[b], outs[b]

      @pl.when(valid(t))
      def _():
        pltpu.make_async_copy(
            tt_hbm.at[pl.ds(0, 32), pl.ds(0, width)], bin_, insem
        ).wait()

      @pl.when(valid(t + 1))
      def _():
        fire_in(t + 1, ins[1 - b])

      @pl.when((t >= 2) & valid(t - 2))
      def _():
        pltpu.make_async_copy(
            bout, t2_hbm.at[pl.ds(0, 32 * _DCOLS)], outsem
        ).wait()

      transpose(bin_, bout)

      @pl.when(valid(t))
      def _():
        u = w + _NUM_WORKERS * t
        pltpu.async_copy(
            bout, t2_hbm.at[pl.ds(u * 32 * _DCOLS, 32 * _DCOLS)], outsem
        )
    return carry

  lax.fori_loop(0, _DSLOTS // 2, pair_body, 0, unroll=False)

  @pl.when(valid(_DSLOTS - 2))
  def _():
    pltpu.make_async_copy(
        outs[0], t2_hbm.at[pl.ds(0, 32 * _DCOLS)], outsem
    ).wait()

  @pl.when(valid(_DSLOTS - 1))
  def _():
    pltpu.make_async_copy(
        outs[1], t2_hbm.at[pl.ds(0, 32 * _DCOLS)], outsem
    ).wait()

  @pl.when(w == 0)
  def _():
    pltpu.sync_copy(tail_hbm, tail_t)
    pltpu.sync_copy(tail_t, t2_hbm.at[pl.ds(_VMAIN * _EMBED // 128, 16)])


@functools.partial(
    pl.kernel,
    out_type=jax.ShapeDtypeStruct((_UNITS * 32 * 128,), jnp.float32),
    mesh=_mesh,
    scratch_types=[
        pltpu.VMEM((_K * 128,), jnp.int32),
        pltpu.VMEM((_K * 128,), jnp.int32),
        pltpu.VMEM((_K * 128, _EMBED), jnp.float32),
        pltpu.VMEM((_K * 128, _EMBED), jnp.float32),
        pltpu.VMEM((_K * 128 * (_EMBED + 1),), jnp.float32),
        pltpu.VMEM((_K * 32 * 128,), jnp.float32),
        pltpu.VMEM((_K * 32 * 128,), jnp.float32),
        pltpu.SemaphoreType.DMA,
        pltpu.SemaphoreType.DMA,
        pltpu.SemaphoreType.DMA,
    ],
    compiler_params=pltpu.CompilerParams(
        use_tc_tiling_on_sc=False, needs_layout_passes=False,
        disable_bounds_checks=True
    ),
)
def _gather(t2_hbm, xa_hbm, out_hbm, idx0, idx1, g0, g1, gp, tr0, tr1,
            isem, gsem, ssem):
  w = _wid()
  lanes = lax.iota(jnp.int32, 16)
  idxs = (idx0, idx1)
  gs = (g0, g1)
  trs = (tr0, tr1)
  mbase = w * (_K * _GSTEPS)

  def fire_idx(s, buf):
    pltpu.async_copy(
        xa_hbm.at[pl.ds((mbase + _K * s) * 128, _K * 128)], buf, isem
    )

  def fire_stores(s, tr):
    m0 = mbase + _K * s
    for r in range(_K):
      m = m0 + r
      ht = m // 256
      bt = (m % 256) // 8
      hr = m % 8
      base = (8 * ht + hr) * 1024 + bt * 8
      for dt in range(4):
        pltpu.async_copy(
            tr.at[pl.ds((r * 32 + 8 * dt) * 128, 1024)],
            out_hbm.at[pl.ds((base + dt * 256) * 128, 1024)],
            ssem,
        )

  fire_idx(0, idxs[0])
  pltpu.make_async_copy(
      xa_hbm.at[pl.ds(0, _K * 128)], idxs[0], isem
  ).wait()
  pltpu.async_copy(t2_hbm.at[idxs[0]], gs[0], gsem)
  fire_idx(1, idxs[1])

  def pair_body(p, carry):
    for b in range(2):
      s = 2 * p + b
      g_v, tr_v = gs[b], trs[b]
      pltpu.make_async_copy(t2_hbm.at[idxs[b]], g_v, gsem).wait()

      @pl.when(s + 1 < _GSTEPS)
      def _():
        pltpu.make_async_copy(
            xa_hbm.at[pl.ds(0, _K * 128)], idxs[1 - b], isem
        ).wait()
        pltpu.async_copy(t2_hbm.at[idxs[1 - b]], gs[1 - b], gsem)

      @pl.when(s + 2 < _GSTEPS)
      def _():
        fire_idx(s + 2, idxs[b])

      @pl.when(s >= 2)
      def _():
        pltpu.make_async_copy(
            tr_v, out_hbm.at[pl.ds(0, _K * 32 * 128)], ssem
        ).wait()

      def rbody(q, carry):
        gp[pl.ds(q * 33, 16)] = g_v[q, pl.ds(0, 16)]
        gp[pl.ds(q * 33 + 16, 16)] = g_v[q, pl.ds(16, 16)]
        return carry

      lax.fori_loop(0, 128 * _K, rbody, 0, unroll=8)

      lanes33 = lanes * 33

      def tbody(i, carry):
        r = i // 8
        av = lanes33 + i * (16 * 33)
        tbase = r * 3968 + i * 16
        for d in range(_EMBED):
          tr_v[pl.ds(tbase + d * 128, 16)] = plsc.load_gather(gp, [av])
          av = av + 1
        return carry

      lax.fori_loop(0, 8 * _K, tbody, 0, unroll=2)
      fire_stores(s, tr_v)
    return carry

  lax.fori_loop(0, _GSTEPS // 2, pair_body, 0, unroll=False)
  pltpu.make_async_copy(
      trs[0], out_hbm.at[pl.ds(0, _K * 32 * 128)], ssem
  ).wait()
  pltpu.make_async_copy(
      trs[1], out_hbm.at[pl.ds(0, _K * 32 * 128)], ssem
  ).wait()


def kernel(x, table):
  batch, hist = x.shape
  tt = jnp.transpose(table)
  xa = (
      jnp.transpose(x)
      .reshape(hist // 8, 8, batch // 128, 128)
      .transpose(0, 2, 1, 3)
      .reshape(_UNITS * 128)
      .astype(jnp.int32)
  )
  tail = lax.slice(table, (_VMAIN, 0), (_VOCAB, _EMBED)).reshape(16, 128)

  t2 = _detile(tt, tail)
  t2v = t2.reshape(_VOCAB, _EMBED)
  a = _gather(t2v, xa)
  out = (
      a.reshape(hist, 4, batch // 128, 8, 128)
      .transpose(2, 4, 0, 1, 3)
      .reshape(batch, hist, _EMBED)
  )
  return out


# speedup vs baseline: 2.1476x; 1.3019x over previous
import functools

import jax
import jax.numpy as jnp
from jax import lax
from jax.experimental import pallas as pl
from jax.experimental.pallas import tpu as pltpu
from jax.experimental.pallas import tpu_sc as plsc

_NUM_CORES = 2
_NUM_SUBCORES = 16
_NUM_WORKERS = _NUM_CORES * _NUM_SUBCORES

_VOCAB = 1000000
_EMBED = 32
_VCOLS = _VOCAB // 128
_VMAIN = _VCOLS * 128
_T2_ROWS = _VOCAB * _EMBED // 128

_DCOLS = 4
_DUNITS = _VCOLS // _DCOLS
_DSLOTS = 62

_UNITS = 6400
_K = 4
_GSTEPS = _UNITS // (_NUM_WORKERS * _K)

_mesh = plsc.VectorSubcoreMesh(core_axis_name="c", subcore_axis_name="s")


def _wid():
  return lax.axis_index("s") * _NUM_CORES + lax.axis_index("c")


@functools.partial(
    pl.kernel,
    out_type=jax.ShapeDtypeStruct((_T2_ROWS, 128), jnp.float32),
    mesh=_mesh,
    scratch_types=[
        pltpu.VMEM((32, 128 * _DCOLS), jnp.float32),
        pltpu.VMEM((32, 128 * _DCOLS), jnp.float32),
        pltpu.VMEM((32 * (128 * _DCOLS + 1),), jnp.float32),
        pltpu.VMEM((32 * _DCOLS, 128), jnp.float32),
        pltpu.VMEM((32 * _DCOLS, 128), jnp.float32),
        pltpu.VMEM((16, 128), jnp.float32),
        pltpu.SemaphoreType.DMA,
        pltpu.SemaphoreType.DMA,
    ],
    compiler_params=pltpu.CompilerParams(
        use_tc_tiling_on_sc=True, needs_layout_passes=False,
        disable_bounds_checks=True
    ),
)
def _detile(tt_hbm, tail_hbm, t2_hbm, in0, in1, inp, out0, out1, tail_t,
            insem, outsem):
  w = _wid()
  lanes = lax.iota(jnp.int32, 16)
  ins = (in0, in1)
  outs = (out0, out1)
  width = 128 * _DCOLS

  def valid(t):
    return (w + _NUM_WORKERS * t) < _DUNITS

  def fire_in(t, buf):
    u = w + _NUM_WORKERS * t
    for dt in range(4):
      pltpu.async_copy(
          tt_hbm.at[pl.ds(dt * 8, 8), pl.ds(u * width, width)],
          buf.at[pl.ds(dt * 8, 8)],
          insem,
      )

  stride = 128 * _DCOLS + 1
  lanes_s = lanes * stride

  def transpose(buf_in, buf_out):
    def rbody(d, carry):
      for i in range(128 * _DCOLS // 16):
        inp[pl.ds(d * stride + 16 * i, 16)] = buf_in[d, pl.ds(16 * i, 16)]
      return carry

    lax.fori_loop(0, 32, rbody, 0, unroll=2)

    def tbody(k, carry):
      av0 = lanes_s + 4 * k
      av1 = av0 + 16 * stride
      for jj in range(4):
        ve = plsc.load_gather(inp, [av0])
        vo = plsc.load_gather(inp, [av1])
        buf_out[k, pl.ds(32 * jj, 16)] = ve
        buf_out[k, pl.ds(32 * jj + 16, 16)] = vo
        av0 = av0 + 1
        av1 = av1 + 1
      return carry

    lax.fori_loop(0, 32 * _DCOLS, tbody, 0, unroll=8)

  fire_in(0, ins[0])

  def pair_body(p, carry):
    for b in range(2):
      t = 2 * p + b
      bin_, bout = ins[b], outs[b]

      @pl.when(valid(t))
      def _():
        pltpu.make_async_copy(
            tt_hbm.at[pl.ds(0, 32), pl.ds(0, width)], bin_, insem
        ).wait()

      @pl.when(valid(t + 1))
      def _():
        fire_in(t + 1, ins[1 - b])

      @pl.when((t >= 2) & valid(t - 2))
      def _():
        pltpu.make_async_copy(
            bout, t2_hbm.at[pl.ds(0, 32 * _DCOLS)], outsem
        ).wait()

      transpose(bin_, bout)

      @pl.when(valid(t))
      def _():
        u = w + _NUM_WORKERS * t
        pltpu.async_copy(
            bout, t2_hbm.at[pl.ds(u * 32 * _DCOLS, 32 * _DCOLS)], outsem
        )
    return carry

  lax.fori_loop(0, _DSLOTS // 2, pair_body, 0, unroll=False)

  @pl.when(valid(_DSLOTS - 2))
  def _():
    pltpu.make_async_copy(
        outs[0], t2_hbm.at[pl.ds(0, 32 * _DCOLS)], outsem
    ).wait()

  @pl.when(valid(_DSLOTS - 1))
  def _():
    pltpu.make_async_copy(
        outs[1], t2_hbm.at[pl.ds(0, 32 * _DCOLS)], outsem
    ).wait()

  @pl.when(w == 0)
  def _():
    pltpu.sync_copy(tail_hbm, tail_t)
    pltpu.sync_copy(tail_t, t2_hbm.at[pl.ds(_VMAIN * _EMBED // 128, 16)])


@functools.partial(
    pl.kernel,
    out_type=jax.ShapeDtypeStruct((_UNITS * 32 * 128,), jnp.float32),
    mesh=_mesh,
    scratch_types=[
        pltpu.VMEM((_K * 128,), jnp.int32),
        pltpu.VMEM((_K * 128,), jnp.int32),
        pltpu.VMEM((_K * 128, _EMBED), jnp.float32),
        pltpu.VMEM((_K * 128, _EMBED), jnp.float32),
        pltpu.VMEM((_K * 128 * (_EMBED + 1),), jnp.float32),
        pltpu.VMEM((_K * 32 * 128,), jnp.float32),
        pltpu.VMEM((_K * 32 * 128,), jnp.float32),
        pltpu.SemaphoreType.DMA,
        pltpu.SemaphoreType.DMA,
        pltpu.SemaphoreType.DMA,
    ],
    compiler_params=pltpu.CompilerParams(
        use_tc_tiling_on_sc=False, needs_layout_passes=False,
        disable_bounds_checks=True
    ),
)
def _gather(t2_hbm, xa_hbm, out_hbm, idx0, idx1, g0, g1, gp, tr0, tr1,
            isem, gsem, ssem):
  w = _wid()
  lanes = lax.iota(jnp.int32, 16)
  idxs = (idx0, idx1)
  gs = (g0, g1)
  trs = (tr0, tr1)
  mbase = w * (_K * _GSTEPS)

  def fire_idx(s, buf):
    pltpu.async_copy(
        xa_hbm.at[pl.ds((mbase + _K * s) * 128, _K * 128)], buf, isem
    )

  def fire_stores(s, tr):
    m0 = mbase + _K * s
    for r in range(_K):
      m = m0 + r
      ht = m // 256
      bt = (m % 256) // 8
      hr = m % 8
      base = (8 * ht + hr) * 1024 + bt * 8
      for dt in range(4):
        pltpu.async_copy(
            tr.at[pl.ds((r * 32 + 8 * dt) * 128, 1024)],
            out_hbm.at[pl.ds((base + dt * 256) * 128, 1024)],
            ssem,
        )

  fire_idx(0, idxs[0])
  pltpu.make_async_copy(
      xa_hbm.at[pl.ds(0, _K * 128)], idxs[0], isem
  ).wait()
  pltpu.async_copy(t2_hbm.at[idxs[0]], gs[0], gsem)
  fire_idx(1, idxs[1])

  def pair_body(p, carry):
    for b in range(2):
      s = 2 * p + b
      g_v, tr_v = gs[b], trs[b]
      pltpu.make_async_copy(t2_hbm.at[idxs[b]], g_v, gsem).wait()

      @pl.when(s + 1 < _GSTEPS)
      def _():
        pltpu.make_async_copy(
            xa_hbm.at[pl.ds(0, _K * 128)], idxs[1 - b], isem
        ).wait()
        pltpu.async_copy(t2_hbm.at[idxs[1 - b]], gs[1 - b], gsem)

      @pl.when(s + 2 < _GSTEPS)
      def _():
        fire_idx(s + 2, idxs[b])

      @pl.when(s >= 2)
      def _():
        pltpu.make_async_copy(
            tr_v, out_hbm.at[pl.ds(0, _K * 32 * 128)], ssem
        ).wait()

      def rbody(q, carry):
        gp[pl.ds(q * 33, 16)] = g_v[q, pl.ds(0, 16)]
        gp[pl.ds(q * 33 + 16, 16)] = g_v[q, pl.ds(16, 16)]
        return carry

      lax.fori_loop(0, 128 * _K, rbody, 0, unroll=8)

      lanes33 = lanes * 33

      def tbody(i, carry):
        r = i // 8
        av0 = lanes33 + i * (16 * 33)
        av1 = av0 + 16
        tbase = r * 3968 + i * 16
        for d in range(16):
          va = plsc.load_gather(gp, [av0])
          vb = plsc.load_gather(gp, [av1])
          tr_v[pl.ds(tbase + d * 128, 16)] = va
          tr_v[pl.ds(tbase + (d + 16) * 128, 16)] = vb
          av0 = av0 + 1
          av1 = av1 + 1
        return carry

      lax.fori_loop(0, 8 * _K, tbody, 0, unroll=2)
      fire_stores(s, tr_v)
    return carry

  lax.fori_loop(0, _GSTEPS // 2, pair_body, 0, unroll=False)
  pltpu.make_async_copy(
      trs[0], out_hbm.at[pl.ds(0, _K * 32 * 128)], ssem
  ).wait()
  pltpu.make_async_copy(
      trs[1], out_hbm.at[pl.ds(0, _K * 32 * 128)], ssem
  ).wait()


def kernel(x, table):
  batch, hist = x.shape
  tt = jnp.transpose(table)
  xa = (
      jnp.transpose(x)
      .reshape(hist // 8, 8, batch // 128, 128)
      .transpose(0, 2, 1, 3)
      .reshape(_UNITS * 128)
      .astype(jnp.int32)
  )
  tail = lax.slice(table, (_VMAIN, 0), (_VOCAB, _EMBED)).reshape(16, 128)

  t2 = _detile(tt, tail)
  t2v = t2.reshape(_VOCAB, _EMBED)
  a = _gather(t2v, xa)
  out = (
      a.reshape(hist, 4, batch // 128, 8, 128)
      .transpose(2, 4, 0, 1, 3)
      .reshape(batch, hist, _EMBED)
  )
  return out


# speedup vs baseline: 3.0553x; 1.4226x over previous
import functools

import jax
import jax.numpy as jnp
from jax import lax
from jax.experimental import pallas as pl
from jax.experimental.pallas import tpu as pltpu
from jax.experimental.pallas import tpu_sc as plsc

_NUM_CORES = 2
_NUM_SUBCORES = 16
_NUM_WORKERS = _NUM_CORES * _NUM_SUBCORES

_VOCAB = 1000000
_EMBED = 32
_VCOLS = _VOCAB // 128
_VMAIN = _VCOLS * 128
_T2_ROWS = _VOCAB * _EMBED // 128

_DCOLS = 4
_DUNITS = _VCOLS // _DCOLS
_DSLOTS = 62

_UNITS = 6400
_K = 4
_GSTEPS = _UNITS // (_NUM_WORKERS * _K)

_mesh = plsc.VectorSubcoreMesh(core_axis_name="c", subcore_axis_name="s")


def _wid():
  return lax.axis_index("s") * _NUM_CORES + lax.axis_index("c")


@functools.partial(
    pl.kernel,
    out_type=jax.ShapeDtypeStruct((_T2_ROWS, 128), jnp.float32),
    mesh=_mesh,
    scratch_types=[
        pltpu.VMEM((32, 128 * _DCOLS), jnp.float32),
        pltpu.VMEM((32, 128 * _DCOLS), jnp.float32),
        pltpu.VMEM((32 * (128 * _DCOLS + 1),), jnp.float32),
        pltpu.VMEM((32 * _DCOLS, 128), jnp.float32),
        pltpu.VMEM((32 * _DCOLS, 128), jnp.float32),
        pltpu.VMEM((16, 128), jnp.float32),
        pltpu.SemaphoreType.DMA,
        pltpu.SemaphoreType.DMA,
    ],
    compiler_params=pltpu.CompilerParams(
        use_tc_tiling_on_sc=True, needs_layout_passes=False,
        disable_bounds_checks=True
    ),
)
def _detile(tt_hbm, tail_hbm, t2_hbm, in0, in1, inp, out0, out1, tail_t,
            insem, outsem):
  w = _wid()
  lanes = lax.iota(jnp.int32, 16)
  ins = (in0, in1)
  outs = (out0, out1)
  width = 128 * _DCOLS

  def valid(t):
    return (w + _NUM_WORKERS * t) < _DUNITS

  def fire_in(t, buf):
    u = w + _NUM_WORKERS * t
    for dt in range(4):
      pltpu.async_copy(
          tt_hbm.at[pl.ds(dt * 8, 8), pl.ds(u * width, width)],
          buf.at[pl.ds(dt * 8, 8)],
          insem,
      )

  stride = 128 * _DCOLS + 1
  lanes_s = lanes * stride

  def transpose(buf_in, buf_out):
    def rbody(d, carry):
      for i in range(0, 128 * _DCOLS // 16, 2):
        va = buf_in[d, pl.ds(16 * i, 16)]
        vb = buf_in[d, pl.ds(16 * i + 16, 16)]
        inp[pl.ds(d * stride + 16 * i, 16)] = va
        inp[pl.ds(d * stride + 16 * i + 16, 16)] = vb
      return carry

    lax.fori_loop(0, 32, rbody, 0, unroll=2)

    def tbody(k, carry):
      av0 = lanes_s + 4 * k
      av1 = av0 + 16 * stride
      for jj in range(4):
        ve = plsc.load_gather(inp, [av0])
        vo = plsc.load_gather(inp, [av1])
        buf_out[k, pl.ds(32 * jj, 16)] = ve
        buf_out[k, pl.ds(32 * jj + 16, 16)] = vo
        av0 = av0 + 1
        av1 = av1 + 1
      return carry

    lax.fori_loop(0, 32 * _DCOLS, tbody, 0, unroll=8)

  fire_in(0, ins[0])

  def pair_body(p, carry):
    for b in range(2):
      t = 2 * p + b
      bin_, bout = ins[b], outs[b]

      @pl.when(valid(t))
      def _():
        pltpu.make_async_copy(
            tt_hbm.at[pl.ds(0, 32), pl.ds(0, width)], bin_, insem
        ).wait()

      @pl.when(valid(t + 1))
      def _():
        fire_in(t + 1, ins[1 - b])

      @pl.when((t >= 2) & valid(t - 2))
      def _():
        pltpu.make_async_copy(
            bout, t2_hbm.at[pl.ds(0, 32 * _DCOLS)], outsem
        ).wait()

      transpose(bin_, bout)

      @pl.when(valid(t))
      def _():
        u = w + _NUM_WORKERS * t
        pltpu.async_copy(
            bout, t2_hbm.at[pl.ds(u * 32 * _DCOLS, 32 * _DCOLS)], outsem
        )
    return carry

  lax.fori_loop(0, _DSLOTS // 2, pair_body, 0, unroll=False)

  @pl.when(valid(_DSLOTS - 2))
  def _():
    pltpu.make_async_copy(
        outs[0], t2_hbm.at[pl.ds(0, 32 * _DCOLS)], outsem
    ).wait()

  @pl.when(valid(_DSLOTS - 1))
  def _():
    pltpu.make_async_copy(
        outs[1], t2_hbm.at[pl.ds(0, 32 * _DCOLS)], outsem
    ).wait()

  @pl.when(w == 0)
  def _():
    pltpu.sync_copy(tail_hbm, tail_t)
    pltpu.sync_copy(tail_t, t2_hbm.at[pl.ds(_VMAIN * _EMBED // 128, 16)])


@functools.partial(
    pl.kernel,
    out_type=jax.ShapeDtypeStruct((_UNITS * 32 * 128,), jnp.float32),
    mesh=_mesh,
    scratch_types=[
        pltpu.VMEM((_K * 128,), jnp.int32),
        pltpu.VMEM((_K * 128,), jnp.int32),
        pltpu.VMEM((_K * 128, _EMBED), jnp.float32),
        pltpu.VMEM((_K * 128, _EMBED), jnp.float32),
        pltpu.VMEM((_K * 128 * (_EMBED + 1),), jnp.float32),
        pltpu.VMEM((_K * 32 * 128,), jnp.float32),
        pltpu.VMEM((_K * 32 * 128,), jnp.float32),
        pltpu.SemaphoreType.DMA,
        pltpu.SemaphoreType.DMA,
        pltpu.SemaphoreType.DMA,
    ],
    compiler_params=pltpu.CompilerParams(
        use_tc_tiling_on_sc=False, needs_layout_passes=False,
        disable_bounds_checks=True
    ),
)
def _gather(t2_hbm, xa_hbm, out_hbm, idx0, idx1, g0, g1, gp, tr0, tr1,
            isem, gsem, ssem):
  w = _wid()
  lanes = lax.iota(jnp.int32, 16)
  idxs = (idx0, idx1)
  gs = (g0, g1)
  trs = (tr0, tr1)
  mbase = w * (_K * _GSTEPS)

  def fire_idx(s, buf):
    pltpu.async_copy(
        xa_hbm.at[pl.ds((mbase + _K * s) * 128, _K * 128)], buf, isem
    )

  def fire_stores(s, tr):
    m0 = mbase + _K * s
    for r in range(_K):
      m = m0 + r
      ht = m // 256
      bt = (m % 256) // 8
      hr = m % 8
      base = (8 * ht + hr) * 1024 + bt * 8
      for dt in range(4):
        pltpu.async_copy(
            tr.at[pl.ds((r * 32 + 8 * dt) * 128, 1024)],
            out_hbm.at[pl.ds((base + dt * 256) * 128, 1024)],
            ssem,
        )

  fire_idx(0, idxs[0])
  pltpu.make_async_copy(
      xa_hbm.at[pl.ds(0, _K * 128)], idxs[0], isem
  ).wait()
  pltpu.async_copy(t2_hbm.at[idxs[0]], gs[0], gsem)
  fire_idx(1, idxs[1])

  def pair_body(p, carry):
    for b in range(2):
      s = 2 * p + b
      g_v, tr_v = gs[b], trs[b]
      pltpu.make_async_copy(t2_hbm.at[idxs[b]], g_v, gsem).wait()

      @pl.when(s + 1 < _GSTEPS)
      def _():
        pltpu.make_async_copy(
            xa_hbm.at[pl.ds(0, _K * 128)], idxs[1 - b], isem
        ).wait()
        pltpu.async_copy(t2_hbm.at[idxs[1 - b]], gs[1 - b], gsem)

      @pl.when(s + 2 < _GSTEPS)
      def _():
        fire_idx(s + 2, idxs[b])

      @pl.when(s >= 2)
      def _():
        pltpu.make_async_copy(
            tr_v, out_hbm.at[pl.ds(0, _K * 32 * 128)], ssem
        ).wait()

      def rbody(q, carry):
        q2 = 2 * q
        a0 = g_v[q2, pl.ds(0, 16)]
        a1 = g_v[q2, pl.ds(16, 16)]
        b0 = g_v[q2 + 1, pl.ds(0, 16)]
        b1 = g_v[q2 + 1, pl.ds(16, 16)]
        gp[pl.ds(q2 * 33, 16)] = a0
        gp[pl.ds(q2 * 33 + 16, 16)] = a1
        gp[pl.ds(q2 * 33 + 33, 16)] = b0
        gp[pl.ds(q2 * 33 + 49, 16)] = b1
        return carry

      lax.fori_loop(0, 64 * _K, rbody, 0, unroll=8)

      lanes33 = lanes * 33

      def tbody(i, carry):
        r = i // 8
        av0 = lanes33 + i * (16 * 33)
        av1 = av0 + 16
        tbase = r * 3968 + i * 16
        for d in range(16):
          va = plsc.load_gather(gp, [av0])
          vb = plsc.load_gather(gp, [av1])
          tr_v[pl.ds(tbase + d * 128, 16)] = va
          tr_v[pl.ds(tbase + (d + 16) * 128, 16)] = vb
          av0 = av0 + 1
          av1 = av1 + 1
        return carry

      lax.fori_loop(0, 8 * _K, tbody, 0, unroll=2)
      fire_stores(s, tr_v)
    return carry

  lax.fori_loop(0, _GSTEPS // 2, pair_body, 0, unroll=False)
  pltpu.make_async_copy(
      trs[0], out_hbm.at[pl.ds(0, _K * 32 * 128)], ssem
  ).wait()
  pltpu.make_async_copy(
      trs[1], out_hbm.at[pl.ds(0, _K * 32 * 128)], ssem
  ).wait()


def kernel(x, table):
  batch, hist = x.shape
  tt = jnp.transpose(table)
  xa = (
      jnp.transpose(x)
      .reshape(hist // 8, 8, batch // 128, 128)
      .transpose(0, 2, 1, 3)
      .reshape(_UNITS * 128)
      .astype(jnp.int32)
  )
  tail = lax.slice(table, (_VMAIN, 0), (_VOCAB, _EMBED)).reshape(16, 128)

  t2 = _detile(tt, tail)
  t2v = t2.reshape(_VOCAB, _EMBED)
  a = _gather(t2v, xa)
  out = (
      a.reshape(hist, 4, batch // 128, 8, 128)
      .transpose(2, 4, 0, 1, 3)
      .reshape(batch, hist, _EMBED)
  )
  return out
